# baseline (speedup 1.0000x reference)
import functools

import jax
import jax.numpy as jnp
from jax import lax
from jax.experimental import pallas as pl
from jax.experimental.pallas import tpu as pltpu
from jax.experimental.pallas import tpu_sc as plsc

VOC = 1000000
EMB = 64
B = 16384

NUM_CORES = 2
NUM_SUBCORES = 16
NUM_WORKERS = NUM_CORES * NUM_SUBCORES
ROWS_PER_W = B // NUM_WORKERS
BLK = 16
PASS_ROWS = 128
NPASS = ROWS_PER_W // PASS_ROWS
BLKS_PER_PASS = PASS_ROWS // BLK

TCH = 256
MAIN_ROWS = (VOC // TCH) * TCH
NCH = MAIN_ROWS // TCH
CH_PAIRS = NCH // (2 * NUM_WORKERS)
TAIL = VOC - MAIN_ROWS

_MESH = plsc.VectorSubcoreMesh(core_axis_name="c", subcore_axis_name="s")
_PARAMS = pltpu.CompilerParams(
    needs_layout_passes=False, use_tc_tiling_on_sc=True)


def _worker_id():
    return lax.axis_index("s") * NUM_CORES + lax.axis_index("c")


def _sc_tr_body(tT_h, out_h, i0_v, i1_v, o0_v, o1_v,
                sem_i0, sem_i1, sem_o0, sem_o1):
    wid = _worker_id()
    iota = lax.iota(jnp.int32, BLK)

    def fire_in(cid, buf_v, sem):
        pltpu.async_copy(
            tT_h.at[:, pl.ds(pl.multiple_of(cid * TCH, TCH), TCH)],
            buf_v, sem)

    def transpose_out(cid, buf_v, sem_i, obuf_v, sem_o):
        pltpu.make_async_copy(tT_h.at[:, pl.ds(0, TCH)], buf_v, sem_i).wait()

        def body(rb, carry):
            roff = pl.ds(pl.multiple_of(rb * BLK, BLK), BLK)
            rows = rb * BLK + iota
            for c in range(EMB):
                v = buf_v[c, roff]
                col = jnp.full((BLK,), c, jnp.int32)
                plsc.store_scatter(obuf_v, [rows, col], v)
            return carry

        lax.fori_loop(0, TCH // BLK, body, 0)
        pltpu.async_copy(
            obuf_v,
            out_h.at[pl.ds(pl.multiple_of(cid * TCH, TCH), TCH)], sem_o)

    def drain_outs():
        dummy = pl.ds(0, TCH)
        pltpu.make_async_copy(out_h.at[dummy], o0_v, sem_o0).wait()
        pltpu.make_async_copy(out_h.at[dummy], o1_v, sem_o1).wait()

    def body(k, carry):
        cid0 = wid * 2 + k * 2 * NUM_WORKERS
        cid1 = cid0 + 1
        fire_in(cid0, i0_v, sem_i0)
        fire_in(cid1, i1_v, sem_i1)

        @pl.when(k > 0)
        def _():
            drain_outs()

        transpose_out(cid0, i0_v, sem_i0, o0_v, sem_o0)
        transpose_out(cid1, i1_v, sem_i1, o1_v, sem_o1)
        return carry

    lax.fori_loop(0, CH_PAIRS, body, 0)

    n_left = NCH - 2 * NUM_WORKERS * CH_PAIRS

    @pl.when(wid < n_left)
    def _():
        cid = 2 * NUM_WORKERS * CH_PAIRS + wid
        fire_in(cid, i0_v, sem_i0)
        drain_outs()
        transpose_out(cid, i0_v, sem_i0, o0_v, sem_o0)

    dummy = pl.ds(0, TCH)
    pltpu.make_async_copy(out_h.at[dummy], o0_v, sem_o0).wait()

    @pl.when(wid >= n_left)
    def _():
        pltpu.make_async_copy(out_h.at[dummy], o1_v, sem_o1).wait()


@functools.partial(
    pl.kernel,
    out_type=jax.ShapeDtypeStruct((MAIN_ROWS, EMB), jnp.float32),
    mesh=_MESH,
    scratch_types=[
        pltpu.VMEM((EMB, TCH), jnp.float32),
        pltpu.VMEM((EMB, TCH), jnp.float32),
        pltpu.VMEM((TCH, EMB), jnp.float32),
        pltpu.VMEM((TCH, EMB), jnp.float32),
        pltpu.SemaphoreType.DMA,
        pltpu.SemaphoreType.DMA,
        pltpu.SemaphoreType.DMA,
        pltpu.SemaphoreType.DMA,
    ],
    compiler_params=_PARAMS,
)
def _sc_transpose(*args):
    _sc_tr_body(*args)


def _sc_dots_body(inpt_h, trgs_h, rand_h, emb_h, ffw_h, etail_h, ftail_h,
                  out_h, rout_h,
                  ii_v, it_v, ir_v,
                  e0_v, c0_v, r0_v, e1_v, c1_v, r1_v, oc_v, or_v,
                  sem_e0, sem_c0, sem_r0, sem_e1, sem_c1, sem_r1):
    base = _worker_id() * ROWS_PER_W

    src = pl.ds(base, ROWS_PER_W)
    pltpu.sync_copy(inpt_h.at[src], ii_v)
    pltpu.sync_copy(trgs_h.at[src], it_v)
    pltpu.sync_copy(rand_h.at[src], ir_v)

    iota = lax.iota(jnp.int32, BLK)
    bufs = ((e0_v, c0_v, r0_v, sem_e0, sem_c0, sem_r0),
            (e1_v, c1_v, r1_v, sem_e1, sem_c1, sem_r1))

    def row_copy(main_h, tail_h, ridx, dst, sem):
        @pl.when(ridx < MAIN_ROWS)
        def _():
            pltpu.async_copy(main_h.at[pl.ds(ridx, 1)], dst, sem)

        @pl.when(ridx >= MAIN_ROWS)
        def _():
            pltpu.async_copy(tail_h.at[pl.ds(ridx - MAIN_ROWS, 1)], dst, sem)

    def fire(p, buf):
        e_v, c_v, r_v, sem_e, sem_c, sem_r = buf
        pbase = p * PASS_ROWS

        def body(rb, carry):
            off = pl.ds(pl.multiple_of(pbase + rb * BLK, BLK), BLK)
            vi = ii_v[off]
            vt = it_v[off]
            vr = ir_v[off]
            for j in range(BLK):
                di = pl.ds(rb * BLK + j, 1)
                row_copy(emb_h, etail_h, vi[j], e_v.at[di], sem_e)
                row_copy(ffw_h, ftail_h, vt[j], c_v.at[di], sem_c)
                row_copy(ffw_h, ftail_h, vr[j], r_v.at[di], sem_r)
            return carry

        lax.fori_loop(0, BLKS_PER_PASS, body, 0)

    def drain_and_reduce(p, buf):
        e_v, c_v, r_v, sem_e, sem_c, sem_r = buf
        pbase = p * PASS_ROWS
        dummy = pl.ds(0, PASS_ROWS)
        pltpu.make_async_copy(emb_h.at[dummy], e_v, sem_e).wait()
        pltpu.make_async_copy(ffw_h.at[dummy], c_v, sem_c).wait()
        pltpu.make_async_copy(ffw_h.at[dummy], r_v, sem_r).wait()

        def body(rb, carry):
            rows = rb * BLK + iota
            acc_c = jnp.zeros((BLK,), jnp.float32)
            acc_r = jnp.zeros((BLK,), jnp.float32)
            for j in range(EMB):
                col = jnp.full((BLK,), j, jnp.int32)
                e = plsc.load_gather(e_v, [rows, col])
                c = plsc.load_gather(c_v, [rows, col])
                r = plsc.load_gather(r_v, [rows, col])
                acc_c = acc_c + c * e
                acc_r = acc_r + r * e
            off = pl.ds(pl.multiple_of(pbase + rb * BLK, BLK), BLK)
            oc_v[off] = acc_c
            or_v[off] = acc_r
            return carry

        lax.fori_loop(0, BLKS_PER_PASS, body, 0)

    fire(0, bufs[0])
    for p in range(NPASS):
        if p + 1 < NPASS:
            fire(p + 1, bufs[(p + 1) % 2])
        drain_and_reduce(p, bufs[p % 2])

    dst = pl.ds(base, ROWS_PER_W)
    pltpu.sync_copy(oc_v, out_h.at[dst])
    pltpu.sync_copy(or_v, rout_h.at[dst])


@functools.partial(
    pl.kernel,
    out_type=(
        jax.ShapeDtypeStruct((B,), jnp.float32),
        jax.ShapeDtypeStruct((B,), jnp.float32),
    ),
    mesh=_MESH,
    scratch_types=[
        pltpu.VMEM((ROWS_PER_W,), jnp.int32),
        pltpu.VMEM((ROWS_PER_W,), jnp.int32),
        pltpu.VMEM((ROWS_PER_W,), jnp.int32),
        pltpu.VMEM((PASS_ROWS, EMB), jnp.float32),
        pltpu.VMEM((PASS_ROWS, EMB), jnp.float32),
        pltpu.VMEM((PASS_ROWS, EMB), jnp.float32),
        pltpu.VMEM((PASS_ROWS, EMB), jnp.float32),
        pltpu.VMEM((PASS_ROWS, EMB), jnp.float32),
        pltpu.VMEM((PASS_ROWS, EMB), jnp.float32),
        pltpu.VMEM((ROWS_PER_W,), jnp.float32),
        pltpu.VMEM((ROWS_PER_W,), jnp.float32),
        pltpu.SemaphoreType.DMA,
        pltpu.SemaphoreType.DMA,
        pltpu.SemaphoreType.DMA,
        pltpu.SemaphoreType.DMA,
        pltpu.SemaphoreType.DMA,
        pltpu.SemaphoreType.DMA,
    ],
    compiler_params=_PARAMS,
)
def _sc_dots(*args):
    _sc_dots_body(*args)


def _loss_body(a_ref, b_ref, o_ref):
    a = a_ref[...]
    b = b_ref[...]
    pa = jnp.clip(jax.nn.sigmoid(a), 1e-07, 1 - 1e-07)
    pb = jnp.clip(jax.nn.sigmoid(b), 1e-07, 1 - 1e-07)
    pst = -jnp.mean(jnp.log(pa))
    ngt = -jnp.mean(jnp.log(1.0 - pb))
    o_ref[0, 0] = pst + ngt


_loss_call = pl.pallas_call(
    _loss_body,
    out_shape=jax.ShapeDtypeStruct((1, 1), jnp.float32),
    out_specs=pl.BlockSpec(memory_space=pltpu.SMEM),
)


def kernel(inpt, trgs, rand, emb_table, ffw_weight):
    inpt = inpt.astype(jnp.int32)
    trgs = trgs.astype(jnp.int32)
    rand = rand[: inpt.shape[0]].astype(jnp.int32)
    emb_rm = _sc_transpose(emb_table.T)
    ffw_rm = _sc_transpose(ffw_weight.T)
    dots, rdots = _sc_dots(
        inpt, trgs, rand, emb_rm, ffw_rm,
        emb_table[MAIN_ROWS:], ffw_weight[MAIN_ROWS:])
    loss = _loss_call(dots.reshape(128, 128), rdots.reshape(128, 128))
    return loss[0, 0]


# speedup vs baseline: 3.6794x; 3.6794x over previous
import functools

import jax
import jax.numpy as jnp
from jax import lax
from jax.experimental import pallas as pl
from jax.experimental.pallas import tpu as pltpu
from jax.experimental.pallas import tpu_sc as plsc

VOC = 1000000
EMB = 64
B = 16384

NUM_CORES = 2
NUM_SUBCORES = 16
NUM_WORKERS = NUM_CORES * NUM_SUBCORES
ROWS_PER_W = B // NUM_WORKERS
BLK = 16
PASS_ROWS = 128
NPASS = ROWS_PER_W // PASS_ROWS
BLKS_PER_PASS = PASS_ROWS // BLK


def _sc_body(inpt_h, trgs_h, rand_h, emb_h, ffw_h, out_h, rout_h,
             ii_v, it_v, ir_v,
             e0_v, c0_v, r0_v, e1_v, c1_v, r1_v, oc_v, or_v,
             sem_e0, sem_c0, sem_r0, sem_e1, sem_c1, sem_r1):
    wid = lax.axis_index("s") * NUM_CORES + lax.axis_index("c")
    base = wid * ROWS_PER_W

    src = pl.ds(base, ROWS_PER_W)
    pltpu.sync_copy(inpt_h.at[src], ii_v)
    pltpu.sync_copy(trgs_h.at[src], it_v)
    pltpu.sync_copy(rand_h.at[src], ir_v)

    iota = lax.iota(jnp.int32, BLK)
    bufs = ((e0_v, c0_v, r0_v, sem_e0, sem_c0, sem_r0),
            (e1_v, c1_v, r1_v, sem_e1, sem_c1, sem_r1))

    def fire(p, buf):
        e_v, c_v, r_v, sem_e, sem_c, sem_r = buf
        pbase = p * PASS_ROWS

        def body(rb, carry):
            off = pl.ds(pl.multiple_of(pbase + rb * BLK, BLK), BLK)
            vi = ii_v[off]
            vt = it_v[off]
            vr = ir_v[off]
            for j in range(BLK):
                di = pl.ds(rb * BLK + j, 1)
                pltpu.async_copy(emb_h.at[pl.ds(vi[j], 1)], e_v.at[di], sem_e)
                pltpu.async_copy(ffw_h.at[pl.ds(vt[j], 1)], c_v.at[di], sem_c)
                pltpu.async_copy(ffw_h.at[pl.ds(vr[j], 1)], r_v.at[di], sem_r)
            return carry

        lax.fori_loop(0, BLKS_PER_PASS, body, 0)

    def drain_and_reduce(p, buf):
        e_v, c_v, r_v, sem_e, sem_c, sem_r = buf
        pbase = p * PASS_ROWS

        dummy = pl.ds(0, PASS_ROWS)
        pltpu.make_async_copy(emb_h.at[dummy], e_v, sem_e).wait()
        pltpu.make_async_copy(ffw_h.at[dummy], c_v, sem_c).wait()
        pltpu.make_async_copy(ffw_h.at[dummy], r_v, sem_r).wait()

        def body(rb, carry):
            rows = rb * BLK + iota
            acc_c = jnp.zeros((BLK,), jnp.float32)
            acc_r = jnp.zeros((BLK,), jnp.float32)
            for j in range(EMB):
                col = jnp.full((BLK,), j, jnp.int32)
                e = plsc.load_gather(e_v, [rows, col])
                c = plsc.load_gather(c_v, [rows, col])
                r = plsc.load_gather(r_v, [rows, col])
                acc_c = acc_c + c * e
                acc_r = acc_r + r * e
            off = pl.ds(pl.multiple_of(pbase + rb * BLK, BLK), BLK)
            oc_v[off] = acc_c
            or_v[off] = acc_r
            return carry

        lax.fori_loop(0, BLKS_PER_PASS, body, 0)

    fire(0, bufs[0])
    for p in range(NPASS):
        if p + 1 < NPASS:
            fire(p + 1, bufs[(p + 1) % 2])
        drain_and_reduce(p, bufs[p % 2])

    dst = pl.ds(base, ROWS_PER_W)
    pltpu.sync_copy(oc_v, out_h.at[dst])
    pltpu.sync_copy(or_v, rout_h.at[dst])


@functools.partial(
    pl.kernel,
    out_type=(
        jax.ShapeDtypeStruct((B,), jnp.float32),
        jax.ShapeDtypeStruct((B,), jnp.float32),
    ),
    mesh=plsc.VectorSubcoreMesh(core_axis_name="c", subcore_axis_name="s"),
    scratch_types=[
        pltpu.VMEM((ROWS_PER_W,), jnp.int32),
        pltpu.VMEM((ROWS_PER_W,), jnp.int32),
        pltpu.VMEM((ROWS_PER_W,), jnp.int32),
        pltpu.VMEM((PASS_ROWS, EMB), jnp.float32),
        pltpu.VMEM((PASS_ROWS, EMB), jnp.float32),
        pltpu.VMEM((PASS_ROWS, EMB), jnp.float32),
        pltpu.VMEM((PASS_ROWS, EMB), jnp.float32),
        pltpu.VMEM((PASS_ROWS, EMB), jnp.float32),
        pltpu.VMEM((PASS_ROWS, EMB), jnp.float32),
        pltpu.VMEM((ROWS_PER_W,), jnp.float32),
        pltpu.VMEM((ROWS_PER_W,), jnp.float32),
        pltpu.SemaphoreType.DMA,
        pltpu.SemaphoreType.DMA,
        pltpu.SemaphoreType.DMA,
        pltpu.SemaphoreType.DMA,
        pltpu.SemaphoreType.DMA,
        pltpu.SemaphoreType.DMA,
    ],
    compiler_params=pltpu.CompilerParams(
        needs_layout_passes=False, use_tc_tiling_on_sc=True),
)
def _sc_dots(*args):
    _sc_body(*args)


def _loss_body(a_ref, b_ref, o_ref):
    a = a_ref[...]
    b = b_ref[...]
    pa = jnp.clip(jax.nn.sigmoid(a), 1e-07, 1 - 1e-07)
    pb = jnp.clip(jax.nn.sigmoid(b), 1e-07, 1 - 1e-07)
    pst = -jnp.mean(jnp.log(pa))
    ngt = -jnp.mean(jnp.log(1.0 - pb))
    o_ref[0, 0] = pst + ngt


_loss_call = pl.pallas_call(
    _loss_body,
    out_shape=jax.ShapeDtypeStruct((1, 1), jnp.float32),
    out_specs=pl.BlockSpec(memory_space=pltpu.SMEM),
)


def kernel(inpt, trgs, rand, emb_table, ffw_weight):
    inpt = inpt.astype(jnp.int32)
    trgs = trgs.astype(jnp.int32)
    rand = rand[: inpt.shape[0]].astype(jnp.int32)
    dots, rdots = _sc_dots(inpt, trgs, rand, emb_table, ffw_weight)
    loss = _loss_call(dots.reshape(128, 128), rdots.reshape(128, 128))
    return loss[0, 0]
